# baseline (speedup 1.0000x reference)
import functools

import jax
import jax.numpy as jnp
from jax import lax
from jax.experimental import pallas as pl
from jax.experimental.pallas import tpu as pltpu
from jax.experimental.pallas import tpu_sc as plsc

_QUEUE_NUM = 1000
_MAX_NUM = 64
_DIM = 256
_BETA = 0.9
_X = 4096
_L = 16
_NW = 32
_QPW = -(-_QUEUE_NUM // _NW)
_QPAD = 1024
_NCH = _DIM // _L
_UNROLL = 8
_BLK = _MAX_NUM * _DIM


def _splat_i32(x):
    return jnp.full((_L,), x, dtype=jnp.int32)


def _sc_confounder_update(vec_flat, order, starts, counts, qlist, bank_flat):
    mesh = plsc.VectorSubcoreMesh(
        core_axis_name="c", subcore_axis_name="s", num_cores=2, num_subcores=16
    )

    @functools.partial(
        pl.kernel,
        out_type=jax.ShapeDtypeStruct((_QUEUE_NUM, _MAX_NUM, _DIM),
                                      jnp.float32),
        mesh=mesh,
        compiler_params=pltpu.CompilerParams(needs_layout_passes=False),
        scratch_types=[
            pltpu.VMEM((2, _MAX_NUM, _DIM), jnp.float32),
            pltpu.VMEM((2, _DIM), jnp.float32),
            pltpu.VMEM((_X,), jnp.int32),
            pltpu.VMEM((_QPAD,), jnp.int32),
            pltpu.VMEM((_QPAD,), jnp.int32),
            pltpu.VMEM((_QPAD,), jnp.int32),
            pltpu.SemaphoreType.DMA,
            pltpu.SemaphoreType.DMA,
            pltpu.SemaphoreType.DMA,
        ],
    )
    def body(vec_hbm, order_hbm, starts_hbm, counts_hbm, qlist_hbm, bank_hbm,
             out_hbm, seq, v, order_v, starts_v, counts_v, qlist_v,
             qsem, osem, vsem):
        wid = lax.axis_index("s") * 2 + lax.axis_index("c")
        iot = lax.iota(jnp.int32, _L)

        pltpu.sync_copy(order_hbm, order_v)
        pltpu.sync_copy(starts_hbm, starts_v)
        pltpu.sync_copy(counts_hbm, counts_v)
        pltpu.sync_copy(qlist_hbm, qlist_v)

        def sload_i32(ref, pos):
            return jnp.max(plsc.load_gather(ref, [_splat_i32(pos)]))

        def my_q(qi):
            return sload_i32(qlist_v, wid * _QPW + qi)

        def qin_copy(qi, q):
            slot = jnp.bitwise_and(qi, 1)
            return pltpu.make_async_copy(
                bank_hbm.at[q], seq.at[slot], qsem)

        def qout_copy(qi, q):
            slot = jnp.bitwise_and(qi, 1)
            return pltpu.make_async_copy(
                seq.at[slot], out_hbm.at[q], osem)

        def v_copy(start, j):
            ii = sload_i32(order_v, start + j)
            slot = jnp.bitwise_and(j, 1)
            return pltpu.make_async_copy(
                vec_hbm.at[ii], v.at[slot], vsem)

        def process_queue(qoff, start, cnt):
            v_copy(start, 0).start()

            def sample_body(j, _):
                v_copy(start, j).wait()

                @pl.when(j + 1 < cnt)
                def _():
                    v_copy(start, j + 1).start()

                vslot = jnp.bitwise_and(j, 1)
                vch = [v[vslot, pl.ds(c * _L, _L)] for c in range(_NCH)]

                def row_group(g, carry):
                    best_n, best_d, best_r = carry
                    for u in range(_UNROLL):
                        r = g * _UNROLL + u
                        acc = jnp.zeros((_L,), jnp.float32)
                        nacc = jnp.zeros((_L,), jnp.float32)
                        for c in range(_NCH):
                            srow = seq[qoff, r, pl.ds(c * _L, _L)]
                            acc = acc + srow * vch[c]
                            nacc = nacc + srow * srow
                        dot = jnp.sum(acc)
                        num = dot * jnp.abs(dot)
                        den = jnp.maximum(jnp.sum(nacc), 1e-30)
                        better = num * best_d > best_n * den
                        best_n = jnp.where(better, num, best_n)
                        best_d = jnp.where(better, den, best_d)
                        best_r = jnp.where(better, r, best_r)
                    return best_n, best_d, best_r

                _, _, best_r = lax.fori_loop(
                    0, _MAX_NUM // _UNROLL, row_group,
                    (jnp.float32(-jnp.inf), jnp.float32(1.0), jnp.int32(0)))

                for c in range(_NCH):
                    old = seq[qoff, best_r, pl.ds(c * _L, _L)]
                    seq[qoff, best_r, pl.ds(c * _L, _L)] = (
                        _BETA * old + (1.0 - _BETA) * vch[c])
                return 0

            lax.fori_loop(0, cnt, sample_body, 0)

        qs0 = plsc.load_gather(qlist_v, [wid * _QPW + iot])
        qs1 = plsc.load_gather(qlist_v, [wid * _QPW + _L + iot])
        nvalid = (plsc.all_reduce_population_count(qs0 < _QUEUE_NUM)
                  + plsc.all_reduce_population_count(qs1 < _QUEUE_NUM))
        nvalid = jnp.max(nvalid)

        @pl.when(nvalid > 0)
        def _():
            qin_copy(0, my_q(0)).start()

        def queue_body(qi, _):
            @pl.when(qi < nvalid)
            def _():
                q = my_q(qi)
                qin_copy(qi, q).wait()

                @pl.when(qi >= 1)
                def _():
                    qout_copy(qi - 1, my_q(qi - 1)).wait()

                @pl.when(qi + 1 < nvalid)
                def _():
                    qin_copy(qi + 1, my_q(qi + 1)).start()

                qoff = jnp.bitwise_and(qi, 1)
                cnt = sload_i32(counts_v, q)
                start = sload_i32(starts_v, q)

                @pl.when(cnt > 0)
                def _():
                    process_queue(qoff, start, cnt)

                qout_copy(qi, q).start()

            return 0

        lax.fori_loop(0, _QPW, queue_body, 0)

        @pl.when(nvalid > 0)
        def _():
            qout_copy(nvalid - 1, my_q(nvalid - 1)).wait()

    return body(vec_flat, order, starts, counts, qlist, bank_flat)


def kernel(vectors, labels, bank):
    labels = labels.astype(jnp.int32)
    slab, order = lax.sort(
        (labels, jnp.arange(_X, dtype=jnp.int32)), num_keys=1)
    qids = jnp.arange(_QUEUE_NUM, dtype=jnp.int32)
    starts = jnp.searchsorted(slab, qids, side="left").astype(jnp.int32)
    ends = jnp.searchsorted(slab, qids, side="right").astype(jnp.int32)
    counts = ends - starts
    by_load = jnp.argsort(-counts, stable=True).astype(jnp.int32)
    p = jnp.arange(_QUEUE_NUM, dtype=jnp.int32)
    blk, pos = p // _NW, p % _NW
    worker = jnp.where(blk % 2 == 0, pos, _NW - 1 - pos)
    qlist = jnp.full((_QPAD,), _QUEUE_NUM, dtype=jnp.int32)
    qlist = qlist.at[worker * _QPW + blk].set(by_load)
    starts = jnp.pad(starts, (0, _QPAD - _QUEUE_NUM))
    counts = jnp.pad(counts, (0, _QPAD - _QUEUE_NUM))

    return _sc_confounder_update(vectors, order, starts, counts, qlist, bank)


# speedup vs baseline: 1.2297x; 1.2297x over previous
import functools

import jax
import jax.numpy as jnp
from jax import lax
from jax.experimental import pallas as pl
from jax.experimental.pallas import tpu as pltpu
from jax.experimental.pallas import tpu_sc as plsc

_QUEUE_NUM = 1000
_MAX_NUM = 64
_DIM = 256
_BETA = 0.9
_X = 4096
_L = 16
_NW = 32
_QPW = -(-_QUEUE_NUM // _NW)
_QPAD = 1024
_NCH = _DIM // _L
_UNROLL = 4
_BLK = _MAX_NUM * _DIM


def _splat_i32(x):
    return jnp.full((_L,), x, dtype=jnp.int32)


def _sc_confounder_update(vec_flat, order, starts, counts, qlist, bank_flat):
    mesh = plsc.VectorSubcoreMesh(
        core_axis_name="c", subcore_axis_name="s", num_cores=2, num_subcores=16
    )

    @functools.partial(
        pl.kernel,
        out_type=jax.ShapeDtypeStruct((_QUEUE_NUM, _MAX_NUM, _DIM),
                                      jnp.float32),
        mesh=mesh,
        compiler_params=pltpu.CompilerParams(needs_layout_passes=False),
        scratch_types=[
            pltpu.VMEM((2, _MAX_NUM, _DIM), jnp.float32),
            pltpu.VMEM((2, _DIM), jnp.float32),
            pltpu.VMEM((_X,), jnp.int32),
            pltpu.VMEM((_QPAD,), jnp.int32),
            pltpu.VMEM((_QPAD,), jnp.int32),
            pltpu.VMEM((_QPAD,), jnp.int32),
            pltpu.SemaphoreType.DMA,
            pltpu.SemaphoreType.DMA,
            pltpu.SemaphoreType.DMA,
        ],
    )
    def body(vec_hbm, order_hbm, starts_hbm, counts_hbm, qlist_hbm, bank_hbm,
             out_hbm, seq, v, order_v, starts_v, counts_v, qlist_v,
             qsem, osem, vsem):
        wid = lax.axis_index("s") * 2 + lax.axis_index("c")
        iot = lax.iota(jnp.int32, _L)

        pltpu.sync_copy(order_hbm, order_v)
        pltpu.sync_copy(starts_hbm, starts_v)
        pltpu.sync_copy(counts_hbm, counts_v)
        pltpu.sync_copy(qlist_hbm, qlist_v)

        def sload_i32(ref, pos):
            return jnp.max(plsc.load_gather(ref, [_splat_i32(pos)]))

        def my_q(qi):
            return sload_i32(qlist_v, wid * _QPW + qi)

        def qin_copy(qi, q):
            slot = jnp.bitwise_and(qi, 1)
            return pltpu.make_async_copy(
                bank_hbm.at[q], seq.at[slot], qsem)

        def qout_copy(qi, q):
            slot = jnp.bitwise_and(qi, 1)
            return pltpu.make_async_copy(
                seq.at[slot], out_hbm.at[q], osem)

        def v_copy(start, j):
            ii = sload_i32(order_v, start + j)
            slot = jnp.bitwise_and(j, 1)
            return pltpu.make_async_copy(
                vec_hbm.at[ii], v.at[slot], vsem)

        def process_queue(qoff, start, cnt):
            v_copy(start, 0).start()

            def sample_body(j, _):
                v_copy(start, j).wait()

                @pl.when(j + 1 < cnt)
                def _():
                    v_copy(start, j + 1).start()

                vslot = jnp.bitwise_and(j, 1)
                vch = [v[vslot, pl.ds(c * _L, _L)] for c in range(_NCH)]

                def row_group(g, carry):
                    best_n, best_d, best_r = carry
                    for u in range(_UNROLL):
                        r = g * _UNROLL + u
                        acc = jnp.zeros((_L,), jnp.float32)
                        nacc = jnp.zeros((_L,), jnp.float32)
                        for c in range(_NCH):
                            srow = seq[qoff, r, pl.ds(c * _L, _L)]
                            acc = acc + srow * vch[c]
                            nacc = nacc + srow * srow
                        dot = jnp.sum(acc)
                        num = dot * jnp.abs(dot)
                        den = jnp.maximum(jnp.sum(nacc), 1e-30)
                        better = num * best_d > best_n * den
                        best_n = jnp.where(better, num, best_n)
                        best_d = jnp.where(better, den, best_d)
                        best_r = jnp.where(better, r, best_r)
                    return best_n, best_d, best_r

                _, _, best_r = lax.fori_loop(
                    0, _MAX_NUM // _UNROLL, row_group,
                    (jnp.float32(-jnp.inf), jnp.float32(1.0), jnp.int32(0)))

                for c in range(_NCH):
                    old = seq[qoff, best_r, pl.ds(c * _L, _L)]
                    seq[qoff, best_r, pl.ds(c * _L, _L)] = (
                        _BETA * old + (1.0 - _BETA) * vch[c])
                return 0

            lax.fori_loop(0, cnt, sample_body, 0)

        qs0 = plsc.load_gather(qlist_v, [wid * _QPW + iot])
        qs1 = plsc.load_gather(qlist_v, [wid * _QPW + _L + iot])
        nvalid = (plsc.all_reduce_population_count(qs0 < _QUEUE_NUM)
                  + plsc.all_reduce_population_count(qs1 < _QUEUE_NUM))
        nvalid = jnp.max(nvalid)

        @pl.when(nvalid > 0)
        def _():
            qin_copy(0, my_q(0)).start()

        def queue_body(qi, _):
            @pl.when(qi < nvalid)
            def _():
                q = my_q(qi)
                qin_copy(qi, q).wait()

                @pl.when(qi >= 1)
                def _():
                    qout_copy(qi - 1, my_q(qi - 1)).wait()

                @pl.when(qi + 1 < nvalid)
                def _():
                    qin_copy(qi + 1, my_q(qi + 1)).start()

                qoff = jnp.bitwise_and(qi, 1)
                cnt = sload_i32(counts_v, q)
                start = sload_i32(starts_v, q)

                @pl.when(cnt > 0)
                def _():
                    process_queue(qoff, start, cnt)

                qout_copy(qi, q).start()

            return 0

        lax.fori_loop(0, _QPW, queue_body, 0)

        @pl.when(nvalid > 0)
        def _():
            qout_copy(nvalid - 1, my_q(nvalid - 1)).wait()

    return body(vec_flat, order, starts, counts, qlist, bank_flat)


def kernel(vectors, labels, bank):
    labels = labels.astype(jnp.int32)
    slab, order = lax.sort(
        (labels, jnp.arange(_X, dtype=jnp.int32)), num_keys=1)
    qids = jnp.arange(_QUEUE_NUM, dtype=jnp.int32)
    starts = jnp.searchsorted(slab, qids, side="left").astype(jnp.int32)
    ends = jnp.searchsorted(slab, qids, side="right").astype(jnp.int32)
    counts = ends - starts
    by_load = jnp.argsort(-counts, stable=True).astype(jnp.int32)
    p = jnp.arange(_QUEUE_NUM, dtype=jnp.int32)
    blk, pos = p // _NW, p % _NW
    worker = jnp.where(blk % 2 == 0, pos, _NW - 1 - pos)
    qlist = jnp.full((_QPAD,), _QUEUE_NUM, dtype=jnp.int32)
    qlist = qlist.at[worker * _QPW + blk].set(by_load)
    starts = jnp.pad(starts, (0, _QPAD - _QUEUE_NUM))
    counts = jnp.pad(counts, (0, _QPAD - _QUEUE_NUM))

    return _sc_confounder_update(vectors, order, starts, counts, qlist, bank)
